# Initial kernel scaffold; baseline (speedup 1.0000x reference)
import jax
import jax.numpy as jnp
from jax.experimental import pallas as pl


def kernel(x, W_enc, W_dec, b_enc, b_dec):
    raise NotImplementedError("write your pallas kernel here")


# speedup vs baseline: 4.9050x; 4.9050x over previous
import functools

import jax
import jax.numpy as jnp
from jax.experimental import pallas as pl

B, T, D_IN, D_SAE, K_PER_T = 512, 4, 768, 8192, 32
K = K_PER_T * T
D = T * D_IN

BM = 256
BS = 1024
MB = 256


def _enc_kernel(x_ref, w_ref, b_ref, out_ref):
    out_ref[...] = (
        jnp.dot(x_ref[...], w_ref[...], preferred_element_type=jnp.float32)
        + b_ref[...]
    )


def _mask_kernel(pre_ref, z_ref):
    pre = pre_ref[...]
    ubits = jax.lax.bitcast_convert_type(pre, jnp.uint32)
    key = jnp.where(
        ubits >= jnp.uint32(0x80000000),
        ~ubits,
        ubits | jnp.uint32(0x80000000),
    )
    t = jnp.zeros((pre.shape[0], 1), jnp.uint32)
    for bit in range(31, -1, -1):
        cand = t | jnp.uint32(1 << bit)
        cnt = jnp.sum((key >= cand).astype(jnp.int32), axis=1, keepdims=True)
        t = jnp.where(cnt >= K, cand, t)
    keep = key >= t
    z_ref[...] = jnp.where(keep & (pre > 0.0), pre, 0.0)


def _dec_kernel(z_ref, w_ref, x_ref, bd_ref, xhat_ref, loss_ref, *, n_k):
    k = pl.program_id(1)

    @pl.when(k == 0)
    def _init():
        xhat_ref[...] = jnp.broadcast_to(bd_ref[...], xhat_ref.shape)

    xhat_ref[...] += jnp.dot(
        z_ref[...], w_ref[...], preferred_element_type=jnp.float32
    )

    @pl.when(k == n_k - 1)
    def _loss():
        diff = xhat_ref[...] - x_ref[...]
        loss_ref[...] = jnp.broadcast_to(jnp.sum(diff * diff), loss_ref.shape)


def _run(x, W_enc, W_dec, b_enc, b_dec, interpret=False):
    x2 = x.reshape(B, D)
    We2 = W_enc.reshape(D, D_SAE)
    Wd2 = W_dec.reshape(D_SAE, D)
    be2 = b_enc.reshape(1, D_SAE)
    bd2 = b_dec.reshape(1, D)

    n_b = B // BM
    n_s = D_SAE // BS

    pre = pl.pallas_call(
        _enc_kernel,
        grid=(n_b, n_s),
        in_specs=[
            pl.BlockSpec((BM, D), lambda i, j: (i, 0)),
            pl.BlockSpec((D, BS), lambda i, j: (0, j)),
            pl.BlockSpec((1, BS), lambda i, j: (0, j)),
        ],
        out_specs=pl.BlockSpec((BM, BS), lambda i, j: (i, j)),
        out_shape=jax.ShapeDtypeStruct((B, D_SAE), jnp.float32),
        interpret=interpret,
    )(x2, We2, be2)

    z = pl.pallas_call(
        _mask_kernel,
        grid=(B // MB,),
        in_specs=[pl.BlockSpec((MB, D_SAE), lambda i: (i, 0))],
        out_specs=pl.BlockSpec((MB, D_SAE), lambda i: (i, 0)),
        out_shape=jax.ShapeDtypeStruct((B, D_SAE), jnp.float32),
        interpret=interpret,
    )(pre)

    n_k = D_SAE // BS
    xhat2, loss_parts = pl.pallas_call(
        functools.partial(_dec_kernel, n_k=n_k),
        grid=(n_b, n_k),
        in_specs=[
            pl.BlockSpec((BM, BS), lambda i, k: (i, k)),
            pl.BlockSpec((BS, D), lambda i, k: (k, 0)),
            pl.BlockSpec((BM, D), lambda i, k: (i, 0)),
            pl.BlockSpec((1, D), lambda i, k: (0, 0)),
        ],
        out_specs=[
            pl.BlockSpec((BM, D), lambda i, k: (i, 0)),
            pl.BlockSpec((1, 1, 128), lambda i, k: (i, 0, 0)),
        ],
        out_shape=[
            jax.ShapeDtypeStruct((B, D), jnp.float32),
            jax.ShapeDtypeStruct((n_b, 1, 128), jnp.float32),
        ],
        interpret=interpret,
    )(z, Wd2, x2, bd2)

    recon_loss = jnp.sum(loss_parts[:, 0, 0]) / jnp.float32(B * T)
    x_hat = xhat2.reshape(B, T, D_IN)
    return (recon_loss, x_hat, z)


def kernel(x, W_enc, W_dec, b_enc, b_dec):
    return _run(x, W_enc, W_dec, b_enc, b_dec)


# speedup vs baseline: 5.4319x; 1.1074x over previous
import functools

import jax
import jax.numpy as jnp
from jax.experimental import pallas as pl

B, T, D_IN, D_SAE, K_PER_T = 512, 4, 768, 8192, 32
K = K_PER_T * T

BS_ENC = 2048
BS_DEC = 1024
MB = 256


def _enc_kernel(x_ref, w_ref, b_ref, out_ref):
    t = pl.program_id(1)
    d = jnp.dot(x_ref[...], w_ref[...], preferred_element_type=jnp.float32)

    @pl.when(t == 0)
    def _init():
        out_ref[...] = d + b_ref[...]

    @pl.when(t != 0)
    def _acc():
        out_ref[...] += d


def _mask_kernel(pre_ref, z_ref):
    pre = pre_ref[...]
    ubits = jax.lax.bitcast_convert_type(pre, jnp.uint32)
    key = jnp.where(
        ubits >= jnp.uint32(0x80000000),
        ~ubits,
        ubits | jnp.uint32(0x80000000),
    )
    t = jnp.zeros((pre.shape[0], 1), jnp.uint32)
    for bit in range(31, -1, -1):
        cand = t | jnp.uint32(1 << bit)
        cnt = jnp.sum((key >= cand).astype(jnp.int32), axis=1, keepdims=True)
        t = jnp.where(cnt >= K, cand, t)
    keep = key >= t
    z_ref[...] = jnp.where(keep & (pre > 0.0), pre, 0.0)


def _dec_kernel(z_ref, w_ref, x_ref, bd_ref, xhat_ref, loss_ref, *, n_k):
    k = pl.program_id(0)
    d = jnp.dot(z_ref[...], w_ref[...], preferred_element_type=jnp.float32)

    @pl.when(k == 0)
    def _init():
        xhat_ref[...] = d + bd_ref[...]

    @pl.when(k != 0)
    def _acc():
        xhat_ref[...] += d

    @pl.when(k == n_k - 1)
    def _loss():
        diff = xhat_ref[...] - x_ref[...]
        loss_ref[...] = jnp.broadcast_to(jnp.sum(diff * diff), loss_ref.shape)


def _run(x, W_enc, W_dec, b_enc, b_dec, interpret=False):
    xT = x.transpose(1, 0, 2)
    be2 = b_enc.reshape(1, D_SAE)

    n_s = D_SAE // BS_ENC
    pre = pl.pallas_call(
        _enc_kernel,
        grid=(n_s, T),
        in_specs=[
            pl.BlockSpec((None, B, D_IN), lambda j, t: (t, 0, 0)),
            pl.BlockSpec((None, D_IN, BS_ENC), lambda j, t: (t, 0, j)),
            pl.BlockSpec((1, BS_ENC), lambda j, t: (0, j)),
        ],
        out_specs=pl.BlockSpec((B, BS_ENC), lambda j, t: (0, j)),
        out_shape=jax.ShapeDtypeStruct((B, D_SAE), jnp.float32),
        interpret=interpret,
    )(xT, W_enc, be2)

    z = pl.pallas_call(
        _mask_kernel,
        grid=(B // MB,),
        in_specs=[pl.BlockSpec((MB, D_SAE), lambda i: (i, 0))],
        out_specs=pl.BlockSpec((MB, D_SAE), lambda i: (i, 0)),
        out_shape=jax.ShapeDtypeStruct((B, D_SAE), jnp.float32),
        interpret=interpret,
    )(pre)

    n_k = D_SAE // BS_DEC
    D = T * D_IN
    Wd2 = W_dec.reshape(D_SAE, D)
    x2 = x.reshape(B, D)
    bd2 = b_dec.reshape(1, D)
    xhat2, loss_parts = pl.pallas_call(
        functools.partial(_dec_kernel, n_k=n_k),
        grid=(n_k,),
        in_specs=[
            pl.BlockSpec((B, BS_DEC), lambda k: (0, k)),
            pl.BlockSpec((BS_DEC, D), lambda k: (k, 0)),
            pl.BlockSpec((B, D), lambda k: (0, 0)),
            pl.BlockSpec((1, D), lambda k: (0, 0)),
        ],
        out_specs=[
            pl.BlockSpec((B, D), lambda k: (0, 0)),
            pl.BlockSpec((8, 128), lambda k: (0, 0)),
        ],
        out_shape=[
            jax.ShapeDtypeStruct((B, D), jnp.float32),
            jax.ShapeDtypeStruct((8, 128), jnp.float32),
        ],
        interpret=interpret,
    )(z, Wd2, x2, bd2)

    recon_loss = loss_parts[0, 0] / jnp.float32(B * T)
    x_hat = xhat2.reshape(B, T, D_IN)
    return (recon_loss, x_hat, z)


def kernel(x, W_enc, W_dec, b_enc, b_dec):
    return _run(x, W_enc, W_dec, b_enc, b_dec)


# speedup vs baseline: 6.4037x; 1.1789x over previous
import functools

import jax
import jax.numpy as jnp
from jax.experimental import pallas as pl

B, T, D_IN, D_SAE, K_PER_T = 512, 4, 768, 8192, 32
K = K_PER_T * T

BS_ENC = 1024
BS_DEC = 512
MB = 256


def _enc_kernel(x_ref, w_ref, b_ref, out_ref):
    acc = b_ref[...]
    for t in range(T):
        acc = acc + jnp.dot(
            x_ref[:, t, :], w_ref[t], preferred_element_type=jnp.float32
        )
    out_ref[...] = acc


def _mask_kernel(pre_ref, z_ref):
    pre = pre_ref[...]
    ubits = jax.lax.bitcast_convert_type(pre, jnp.uint32)
    key = jnp.where(
        ubits >= jnp.uint32(0x80000000),
        ~ubits,
        ubits | jnp.uint32(0x80000000),
    )
    t = jnp.zeros((pre.shape[0], 1), jnp.uint32)
    for bit in range(31, -1, -1):
        cand = t | jnp.uint32(1 << bit)
        cnt = jnp.sum((key >= cand).astype(jnp.int32), axis=1, keepdims=True)
        t = jnp.where(cnt >= K, cand, t)
    keep = key >= t
    z_ref[...] = jnp.where(keep & (pre > 0.0), pre, 0.0)


def _dec_kernel(z_ref, w_ref, x_ref, bd_ref, xhat_ref, loss_ref, *, n_k):
    k = pl.program_id(0)
    for t in range(T):
        d = jnp.dot(z_ref[...], w_ref[:, t, :], preferred_element_type=jnp.float32)

        @pl.when(k == 0)
        def _init():
            xhat_ref[:, t, :] = d + bd_ref[t][None, :]

        @pl.when(k != 0)
        def _acc():
            xhat_ref[:, t, :] += d

    @pl.when(k == n_k - 1)
    def _loss():
        diff = xhat_ref[...] - x_ref[...]
        loss_ref[...] = jnp.broadcast_to(jnp.sum(diff * diff), loss_ref.shape)


def _run(x, W_enc, W_dec, b_enc, b_dec, interpret=False):
    be2 = b_enc.reshape(1, D_SAE)

    n_s = D_SAE // BS_ENC
    pre = pl.pallas_call(
        _enc_kernel,
        grid=(n_s,),
        in_specs=[
            pl.BlockSpec((B, T, D_IN), lambda j: (0, 0, 0)),
            pl.BlockSpec((T, D_IN, BS_ENC), lambda j: (0, 0, j)),
            pl.BlockSpec((1, BS_ENC), lambda j: (0, j)),
        ],
        out_specs=pl.BlockSpec((B, BS_ENC), lambda j: (0, j)),
        out_shape=jax.ShapeDtypeStruct((B, D_SAE), jnp.float32),
        interpret=interpret,
    )(x, W_enc, be2)

    z = pl.pallas_call(
        _mask_kernel,
        grid=(B // MB,),
        in_specs=[pl.BlockSpec((MB, D_SAE), lambda i: (i, 0))],
        out_specs=pl.BlockSpec((MB, D_SAE), lambda i: (i, 0)),
        out_shape=jax.ShapeDtypeStruct((B, D_SAE), jnp.float32),
        interpret=interpret,
    )(pre)

    n_k = D_SAE // BS_DEC
    xhat, loss_parts = pl.pallas_call(
        functools.partial(_dec_kernel, n_k=n_k),
        grid=(n_k,),
        in_specs=[
            pl.BlockSpec((B, BS_DEC), lambda k: (0, k)),
            pl.BlockSpec((BS_DEC, T, D_IN), lambda k: (k, 0, 0)),
            pl.BlockSpec((B, T, D_IN), lambda k: (0, 0, 0)),
            pl.BlockSpec((T, D_IN), lambda k: (0, 0)),
        ],
        out_specs=[
            pl.BlockSpec((B, T, D_IN), lambda k: (0, 0, 0)),
            pl.BlockSpec((8, 128), lambda k: (0, 0)),
        ],
        out_shape=[
            jax.ShapeDtypeStruct((B, T, D_IN), jnp.float32),
            jax.ShapeDtypeStruct((8, 128), jnp.float32),
        ],
        interpret=interpret,
    )(z, W_dec, x, b_dec)

    recon_loss = loss_parts[0, 0] / jnp.float32(B * T)
    return (recon_loss, xhat, z)


def kernel(x, W_enc, W_dec, b_enc, b_dec):
    return _run(x, W_enc, W_dec, b_enc, b_dec)
